# baseline (speedup 1.0000x reference)
import functools

import jax
import jax.numpy as jnp
from jax import lax
from jax.experimental import pallas as pl
from jax.experimental.pallas import tpu as pltpu
from jax.experimental.pallas import tpu_sc as plsc

_VB = 8192


def _gather_fn(nc, ns, b_per_w, table_hbm, idx_hbm, out_hbm, idx_v, rows_v, sem):
    wid = lax.axis_index("s") * nc + lax.axis_index("c")
    base = wid * b_per_w
    pltpu.sync_copy(idx_hbm.at[pl.ds(base, b_per_w)], idx_v)
    pltpu.async_copy(table_hbm.at[idx_v], rows_v, sem).wait()
    pltpu.sync_copy(rows_v, out_hbm.at[pl.ds(base, b_per_w)])


def _sc_gather(table, idx_flat):
    info = plsc.get_sparse_core_info()
    nc, ns = info.num_cores, info.num_subcores
    nw = nc * ns
    b_total, d = idx_flat.shape[0], table.shape[1]
    b_per_w = b_total // nw
    mesh = plsc.VectorSubcoreMesh(core_axis_name="c", subcore_axis_name="s")
    kern = functools.partial(
        pl.kernel,
        mesh=mesh,
        out_type=jax.ShapeDtypeStruct((b_total, d), jnp.float32),
        scratch_types=[
            pltpu.VMEM((b_per_w,), jnp.int32),
            pltpu.VMEM((b_per_w, d), jnp.float32),
            pltpu.SemaphoreType.DMA,
        ],
        compiler_params=pltpu.CompilerParams(use_tc_tiling_on_sc=False),
    )(functools.partial(_gather_fn, nc, ns, b_per_w))
    return kern(table, idx_flat)


def _proj_body(h_ref, w_ref, b_ref, o_ref):
    o_ref[...] = (
        jnp.dot(h_ref[...], w_ref[...], preferred_element_type=jnp.float32)
        + b_ref[...]
    )


def _tc_project(hidden, proj_w, proj_b2d):
    r, h = hidden.shape
    v = proj_w.shape[1]
    grid = (pl.cdiv(v, _VB),)
    return pl.pallas_call(
        _proj_body,
        grid=grid,
        in_specs=[
            pl.BlockSpec((r, h), lambda j: (0, 0)),
            pl.BlockSpec((h, _VB), lambda j: (0, j)),
            pl.BlockSpec((1, _VB), lambda j: (0, j)),
        ],
        out_specs=pl.BlockSpec((r, _VB), lambda j: (0, j)),
        out_shape=jax.ShapeDtypeStruct((r, v), jnp.float32),
    )(hidden, proj_w, proj_b2d)


def kernel(input_ids, emb_table, proj_w, proj_b):
    b, q = input_ids.shape
    v = proj_w.shape[1]
    idx_flat = input_ids.reshape(b * q).astype(jnp.int32)
    hidden = _sc_gather(emb_table, idx_flat)
    logits = _tc_project(hidden, proj_w, proj_b.reshape(1, v))
    return logits.reshape(b, q, v)
